# baseline (speedup 1.0000x reference)
import jax
import jax.numpy as jnp
from jax import lax
from jax.experimental import pallas as pl
from jax.experimental.pallas import tpu as pltpu
from jax.experimental.pallas import tpu_sc as plsc

N = 10000
E = 320000
D = 128
G = 128
DIM_CELL = 954

NW = 32
BIGCH = 128
NBCH = 80
EPW = NBCH * BIGCH
EPAD = NW * EPW

R = 1024
NBLK = 10
NPAD = NBLK * R
RPT = 632
NACC = 16 * RPT

F32 = jnp.float32
I32 = jnp.int32
EPS = 1e-5


def _mesh():
    return plsc.VectorSubcoreMesh(core_axis_name="c", subcore_axis_name="s",
                                  num_cores=2, num_subcores=16)


def _make_agg():
    out_type = jax.ShapeDtypeStruct((2, NPAD, 128), F32)
    scratch = [
        pltpu.VMEM((BIGCH,), I32),
        pltpu.VMEM((BIGCH,), I32),
        pltpu.VMEM((BIGCH, 128), F32),
        pltpu.VMEM_SHARED((NACC, 128), F32),
        pltpu.SemaphoreType.DMA,
    ]

    def body(x_hbm, src_hbm, dst_hbm, agg_out, sidx, didx, rows, acc, sem):
        c = lax.axis_index("c")
        s = lax.axis_index("s")
        w = c * 16 + s

        zv = jnp.zeros((16,), F32)

        def zrow(i, _):
            for cc in range(8):
                rows[i, pl.ds(cc * 16, 16)] = zv
            return 0

        lax.fori_loop(0, BIGCH, zrow, 0)

        base = s * RPT
        nfull = RPT // BIGCH
        remr = RPT - nfull * BIGCH
        for j in range(nfull):
            pltpu.sync_copy(rows, acc.at[pl.ds(base + j * BIGCH, BIGCH)])
        pltpu.sync_copy(rows.at[pl.ds(0, remr)],
                        acc.at[pl.ds(base + nfull * BIGCH, remr)])

        plsc.subcore_barrier()

        def edge_chunk(t, _):
            eb = (w * NBCH + t) * BIGCH
            pltpu.sync_copy(src_hbm.at[pl.ds(eb, BIGCH)], sidx)
            pltpu.sync_copy(dst_hbm.at[pl.ds(eb, BIGCH)], didx)
            pltpu.async_copy(x_hbm.at[sidx], rows, sem).wait()
            pltpu.sync_copy(rows, acc.at[didx], add=True)
            return 0

        lax.fori_loop(0, NBCH, edge_chunk, 0)

        plsc.subcore_barrier()

        ob = s * RPT
        pltpu.sync_copy(acc.at[pl.ds(ob, RPT)], agg_out.at[c, pl.ds(ob, RPT)])

    return pl.kernel(body, out_type=out_type, mesh=_mesh(),
                     scratch_types=scratch)


_agg = _make_agg()


def _segmax(h1, y2, bounds):
    out_type = jax.ShapeDtypeStruct((3 * G * 128,), F32)
    scratch = [
        pltpu.VMEM((16 * NW,), I32),
        pltpu.VMEM((16,), I32),
        pltpu.VMEM((16, 128), F32),
        pltpu.VMEM((16, 128), F32),
        pltpu.VMEM((4, 128), F32),
        pltpu.SemaphoreType.DMA,
    ]

    def body(h1_hbm, y2_hbm, bnd_hbm, out_hbm, bnd_v, idx_v, rb1, rb2, accb,
             sem):
        c = lax.axis_index("c")
        s = lax.axis_index("s")
        w = c * 16 + s
        pltpu.sync_copy(bnd_hbm, bnd_v)
        bvec = bnd_v[pl.ds(w * 16, 16)]
        lanes = lax.broadcasted_iota(I32, (16,), 0)

        for j in range(4):
            g = w * 4 + j
            st = bvec[j]
            en = bvec[8 + j]
            nch = (en - st + 15) // 16

            carry0 = tuple([jnp.full((16,), -jnp.inf, F32)] * 16
                           + [jnp.full((16,), jnp.inf, F32)] * 8)

            def chunk(jj, carry, st=st, en=en):
                iv = jnp.minimum(jnp.full((16,), st, I32) + jj * 16 + lanes,
                                 jnp.full((16,), en - 1, I32))
                idx_v[...] = iv
                pltpu.async_copy(h1_hbm.at[idx_v], rb1, sem).wait()
                pltpu.async_copy(y2_hbm.at[idx_v], rb2, sem).wait()
                out = list(carry)
                for k in range(16):
                    for cc in range(8):
                        v1 = rb1[k, pl.ds(cc * 16, 16)]
                        v2 = rb2[k, pl.ds(cc * 16, 16)]
                        out[cc] = jnp.maximum(out[cc], v1)
                        out[8 + cc] = jnp.maximum(out[8 + cc], v2)
                        out[16 + cc] = jnp.minimum(out[16 + cc], v2)
                return tuple(out)

            res = lax.fori_loop(0, nch, chunk, carry0)
            for cc in range(8):
                accb[0, pl.ds(cc * 16, 16)] = res[cc]
                accb[1, pl.ds(cc * 16, 16)] = res[8 + cc]
                accb[2, pl.ds(cc * 16, 16)] = res[16 + cc]
            for a in range(3):
                pltpu.sync_copy(accb.at[a],
                                out_hbm.at[pl.ds((a * G + g) * 128, 128)])

    fn = pl.kernel(body, out_type=out_type, mesh=_mesh(),
                   scratch_types=scratch)
    return fn(h1, y2, bounds).reshape(3, G, 128)


def _dot(a, b):
    return jnp.dot(a, b, preferred_element_type=F32,
                   precision=lax.Precision.HIGHEST)


def _tca_body(x_ref, agg_ref, w1_ref, b1_ref, w2_ref, b2_ref, y_ref, st_ref):
    i = pl.program_id(0)
    t = x_ref[...] + agg_ref[0] + agg_ref[1]
    h = jnp.maximum(_dot(t, w1_ref[...]) + b1_ref[...], 0.0)
    y = jnp.maximum(_dot(h, w2_ref[...]) + b2_ref[...], 0.0)
    y_ref[...] = y
    rowid = i * R + lax.broadcasted_iota(I32, (R, 1), 0)
    ym = jnp.where(rowid < N, y, 0.0)
    s0 = jnp.sum(ym, axis=0, keepdims=True)
    s1 = jnp.sum(ym * ym, axis=0, keepdims=True)
    upd = jnp.concatenate([s0, s1, jnp.zeros((6, 128), F32)], axis=0)

    @pl.when(i == 0)
    def _():
        st_ref[...] = jnp.zeros_like(st_ref)

    st_ref[...] += upd


def _bn_affine(st_ref, g_ref, b_ref):
    mu = st_ref[0:1, :] / N
    var = st_ref[1:2, :] / N - mu * mu
    a = g_ref[...] / jnp.sqrt(var + EPS)
    return a, b_ref[...] - mu * a


def _tcn_body(y_ref, st_ref, g_ref, b_ref, h_ref):
    a, cb = _bn_affine(st_ref, g_ref, b_ref)
    h_ref[...] = a * y_ref[...] + cb


def _tcb_body(h1_ref, agg_ref, w1_ref, b1_ref, w2_ref, b2_ref, ib_ref,
              y2_ref, st2_ref, bnd_ref):
    i = pl.program_id(0)
    t2 = h1_ref[...] + agg_ref[0] + agg_ref[1]
    h = jnp.maximum(_dot(t2, w1_ref[...]) + b1_ref[...], 0.0)
    y = jnp.maximum(_dot(h, w2_ref[...]) + b2_ref[...], 0.0)
    y2_ref[...] = y
    rowid = i * R + lax.broadcasted_iota(I32, (R, 1), 0)
    valid = rowid < N
    ym = jnp.where(valid, y, 0.0)
    s0 = jnp.sum(ym, axis=0, keepdims=True)
    s1 = jnp.sum(ym * ym, axis=0, keepdims=True)
    upd = jnp.concatenate([s0, s1, jnp.zeros((6, 128), F32)], axis=0)

    ib = ib_ref[0, 0, :][:, None]
    gi = lax.broadcasted_iota(I32, (R, G), 1)
    one = jnp.ones((R, G), I32)
    zero = jnp.zeros((R, G), I32)
    lt = jnp.sum(jnp.where((ib < gi) & valid, one, zero), axis=0,
                 keepdims=True)
    le = jnp.sum(jnp.where((ib <= gi) & valid, one, zero), axis=0,
                 keepdims=True)
    bupd = jnp.concatenate([lt, le, jnp.zeros((6, 128), I32)], axis=0)

    @pl.when(i == 0)
    def _():
        st2_ref[...] = jnp.zeros_like(st2_ref)
        bnd_ref[...] = jnp.zeros_like(bnd_ref)

    st2_ref[...] += upd
    bnd_ref[...] += bupd


def _tcc_body(st2_ref, g2g_ref, g2b_ref, mm_ref,
              ge_ref, c1w_ref, c1b_ref, cbg_ref, cbb_ref, c2w_ref, c2b_ref,
              xd_ref, xc_ref):
    a2, c2 = _bn_affine(st2_ref, g2g_ref, g2b_ref)
    xd1 = mm_ref[0]
    xd2 = jnp.where(a2 > 0, a2 * mm_ref[1] + c2, a2 * mm_ref[2] + c2)
    xd_ref[...] = jnp.concatenate([xd1, xd2], axis=1)

    xc = jnp.tanh(_dot(ge_ref[...], c1w_ref[...]) + c1b_ref[...])
    mu = jnp.mean(xc, axis=0, keepdims=True)
    var = jnp.mean(xc * xc, axis=0, keepdims=True) - mu * mu
    xb = (xc - mu) / jnp.sqrt(var + EPS) * cbg_ref[...] + cbb_ref[...]
    xc_ref[...] = jnp.maximum(_dot(xb, c2w_ref[...]) + c2b_ref[...], 0.0)


def _full(shape):
    return pl.BlockSpec(shape, lambda *_: tuple(0 for _ in shape))


_tca = pl.pallas_call(
    _tca_body,
    grid=(NBLK,),
    in_specs=[
        pl.BlockSpec((R, 128), lambda i: (i, 0)),
        pl.BlockSpec((2, R, 128), lambda i: (0, i, 0)),
        _full((128, 128)), _full((1, 128)), _full((128, 128)), _full((1, 128)),
    ],
    out_specs=[
        pl.BlockSpec((R, 128), lambda i: (i, 0)),
        _full((8, 128)),
    ],
    out_shape=[
        jax.ShapeDtypeStruct((NPAD, 128), F32),
        jax.ShapeDtypeStruct((8, 128), F32),
    ],
)

_tcn = pl.pallas_call(
    _tcn_body,
    grid=(NBLK,),
    in_specs=[
        pl.BlockSpec((R, 128), lambda i: (i, 0)),
        _full((8, 128)), _full((1, 128)), _full((1, 128)),
    ],
    out_specs=pl.BlockSpec((R, 128), lambda i: (i, 0)),
    out_shape=jax.ShapeDtypeStruct((NPAD, 128), F32),
)

_tcb = pl.pallas_call(
    _tcb_body,
    grid=(NBLK,),
    in_specs=[
        pl.BlockSpec((R, 128), lambda i: (i, 0)),
        pl.BlockSpec((2, R, 128), lambda i: (0, i, 0)),
        _full((128, 128)), _full((1, 128)), _full((128, 128)), _full((1, 128)),
        pl.BlockSpec((1, 1, R), lambda i: (i, 0, 0)),
    ],
    out_specs=[
        pl.BlockSpec((R, 128), lambda i: (i, 0)),
        _full((8, 128)),
        _full((8, 128)),
    ],
    out_shape=[
        jax.ShapeDtypeStruct((NPAD, 128), F32),
        jax.ShapeDtypeStruct((8, 128), F32),
        jax.ShapeDtypeStruct((8, 128), I32),
    ],
)

_tcc = pl.pallas_call(
    _tcc_body,
    in_specs=[
        _full((8, 128)), _full((1, 128)), _full((1, 128)),
        _full((3, G, 128)),
        _full((G, 1024)), _full((1024, 128)), _full((1, 128)),
        _full((1, 128)), _full((1, 128)), _full((128, 128)), _full((1, 128)),
    ],
    out_specs=[_full((G, 256)), _full((G, 128))],
    out_shape=[
        jax.ShapeDtypeStruct((G, 256), F32),
        jax.ShapeDtypeStruct((G, 128), F32),
    ],
)


def kernel(drug_feature, drug_adj, ibatch, gexpr_data,
           g1_W1, g1_b1, g1_W2, g1_b2, g1_gamma, g1_beta,
           g2_W1, g2_b1, g2_W2, g2_b2, g2_gamma, g2_beta,
           c1_W, c1_b, cbn_gamma, cbn_beta, c2_W, c2_b):
    src = drug_adj[0].astype(I32)
    dst = drug_adj[1].astype(I32)
    src_p = jnp.concatenate([src, jnp.zeros((EPAD - E,), I32)])
    dst_p = jnp.concatenate([dst, jnp.full((EPAD - E,), N, I32)])

    x_pad = jnp.zeros((NPAD, 128), F32).at[:N].set(drug_feature)

    agg1 = _agg(drug_feature, src_p, dst_p)

    r2 = lambda v: v.reshape(1, 128)
    y1, st1 = _tca(x_pad, agg1, g1_W1, r2(g1_b1), g1_W2, r2(g1_b2))

    h1 = _tcn(y1, st1, r2(g1_gamma), r2(g1_beta))

    agg2 = _agg(h1, src_p, dst_p)

    ib_pad = jnp.concatenate(
        [ibatch.astype(I32), jnp.full((NPAD - N,), G, I32)]).reshape(NBLK, 1, R)
    y2, st2, bnd = _tcb(h1, agg2, g2_W1, r2(g2_b1), g2_W2, r2(g2_b2), ib_pad)

    bw = (jnp.zeros((NW, 16), I32)
          .at[:, 0:4].set(bnd[0].reshape(NW, 4))
          .at[:, 8:12].set(bnd[1].reshape(NW, 4)))
    mm = _segmax(h1, y2, bw.reshape(-1))

    ge_pad = jnp.zeros((G, 1024), F32).at[:, :DIM_CELL].set(gexpr_data)
    c1w_pad = jnp.zeros((1024, 128), F32).at[:DIM_CELL].set(c1_W)

    x_drug, x_cell = _tcc(st2, r2(g2_gamma), r2(g2_beta), mm,
                          ge_pad, c1w_pad, r2(c1_b),
                          r2(cbn_gamma), r2(cbn_beta), c2_W, r2(c2_b))
    return (x_drug, x_cell)


# speedup vs baseline: 1.0014x; 1.0014x over previous
import jax
import jax.numpy as jnp
from jax import lax
from jax.experimental import pallas as pl
from jax.experimental.pallas import tpu as pltpu
from jax.experimental.pallas import tpu_sc as plsc

N = 10000
E = 320000
D = 128
G = 128
DIM_CELL = 954

NW = 32
BIGCH = 128
NBCH = 80
EPW = NBCH * BIGCH
EPAD = NW * EPW

R = 1024
NBLK = 10
NPAD = NBLK * R
RPT = 632
NACC = 16 * RPT

F32 = jnp.float32
I32 = jnp.int32
EPS = 1e-5


def _mesh():
    return plsc.VectorSubcoreMesh(core_axis_name="c", subcore_axis_name="s",
                                  num_cores=2, num_subcores=16)


def _make_agg():
    out_type = jax.ShapeDtypeStruct((2, NPAD, 128), F32)
    scratch = [
        pltpu.VMEM((BIGCH,), I32),
        pltpu.VMEM((BIGCH,), I32),
        pltpu.VMEM((BIGCH, 128), F32),
        pltpu.VMEM_SHARED((NACC, 128), F32),
        pltpu.SemaphoreType.DMA,
    ]

    def body(x_hbm, src_hbm, dst_hbm, agg_out, sidx, didx, rows, acc, sem):
        c = lax.axis_index("c")
        s = lax.axis_index("s")
        w = c * 16 + s

        zv = jnp.zeros((16,), F32)

        def zrow(i, _):
            for cc in range(8):
                rows[i, pl.ds(cc * 16, 16)] = zv
            return 0

        lax.fori_loop(0, BIGCH, zrow, 0)

        base = s * RPT
        nfull = RPT // BIGCH
        remr = RPT - nfull * BIGCH
        for j in range(nfull):
            pltpu.sync_copy(rows, acc.at[pl.ds(base + j * BIGCH, BIGCH)])
        pltpu.sync_copy(rows.at[pl.ds(0, remr)],
                        acc.at[pl.ds(base + nfull * BIGCH, remr)])

        plsc.subcore_barrier()

        def edge_chunk(t, _):
            eb = (w * NBCH + t) * BIGCH
            pltpu.sync_copy(src_hbm.at[pl.ds(eb, BIGCH)], sidx)
            pltpu.sync_copy(dst_hbm.at[pl.ds(eb, BIGCH)], didx)
            pltpu.async_copy(x_hbm.at[sidx], rows, sem).wait()
            pltpu.sync_copy(rows, acc.at[didx], add=True)
            return 0

        lax.fori_loop(0, NBCH, edge_chunk, 0)

        plsc.subcore_barrier()

        ob = s * RPT
        pltpu.sync_copy(acc.at[pl.ds(ob, RPT)], agg_out.at[c, pl.ds(ob, RPT)])

    return pl.kernel(body, out_type=out_type, mesh=_mesh(),
                     scratch_types=scratch)


_agg = _make_agg()


def _segmax(h1, y2, bounds):
    out_type = jax.ShapeDtypeStruct((3 * G * 128,), F32)
    scratch = [
        pltpu.VMEM((16 * NW,), I32),
        pltpu.VMEM((16,), I32),
        pltpu.VMEM((16, 128), F32),
        pltpu.VMEM((16, 128), F32),
        pltpu.VMEM((4, 128), F32),
        pltpu.SemaphoreType.DMA,
    ]

    def body(h1_hbm, y2_hbm, bnd_hbm, out_hbm, bnd_v, idx_v, rb1, rb2, accb,
             sem):
        c = lax.axis_index("c")
        s = lax.axis_index("s")
        w = c * 16 + s
        pltpu.sync_copy(bnd_hbm, bnd_v)
        bvec = bnd_v[pl.ds(w * 16, 16)]
        lanes = lax.broadcasted_iota(I32, (16,), 0)

        for j in range(4):
            g = w * 4 + j
            st = bvec[j]
            en = bvec[8 + j]
            nch = (en - st + 15) // 16

            carry0 = tuple([jnp.full((16,), -jnp.inf, F32)] * 16
                           + [jnp.full((16,), jnp.inf, F32)] * 8)

            def chunk(jj, carry, st=st, en=en):
                iv = jnp.minimum(jnp.full((16,), st, I32) + jj * 16 + lanes,
                                 jnp.full((16,), en - 1, I32))
                idx_v[...] = iv
                pltpu.async_copy(h1_hbm.at[idx_v], rb1, sem).wait()
                pltpu.async_copy(y2_hbm.at[idx_v], rb2, sem).wait()
                out = list(carry)
                for k in range(16):
                    for cc in range(8):
                        v1 = rb1[k, pl.ds(cc * 16, 16)]
                        v2 = rb2[k, pl.ds(cc * 16, 16)]
                        out[cc] = jnp.maximum(out[cc], v1)
                        out[8 + cc] = jnp.maximum(out[8 + cc], v2)
                        out[16 + cc] = jnp.minimum(out[16 + cc], v2)
                return tuple(out)

            res = lax.fori_loop(0, nch, chunk, carry0)
            for cc in range(8):
                accb[0, pl.ds(cc * 16, 16)] = res[cc]
                accb[1, pl.ds(cc * 16, 16)] = res[8 + cc]
                accb[2, pl.ds(cc * 16, 16)] = res[16 + cc]
            for a in range(3):
                pltpu.sync_copy(accb.at[a],
                                out_hbm.at[pl.ds((a * G + g) * 128, 128)])

    fn = pl.kernel(body, out_type=out_type, mesh=_mesh(),
                   scratch_types=scratch)
    return fn(h1, y2, bounds).reshape(3, G, 128)


def _dot(a, b):
    return jnp.dot(a, b, preferred_element_type=F32,
                   precision=lax.Precision.HIGHEST)


def _tca_body(x_ref, agg_ref, w1_ref, b1_ref, w2_ref, b2_ref, y_ref, st_ref):
    i = pl.program_id(0)
    t = x_ref[...] + agg_ref[0] + agg_ref[1]
    h = jnp.maximum(_dot(t, w1_ref[...]) + b1_ref[...], 0.0)
    y = jnp.maximum(_dot(h, w2_ref[...]) + b2_ref[...], 0.0)
    y_ref[...] = y
    rowid = i * R + lax.broadcasted_iota(I32, (R, 1), 0)
    ym = jnp.where(rowid < N, y, 0.0)
    s0 = jnp.sum(ym, axis=0, keepdims=True)
    s1 = jnp.sum(ym * ym, axis=0, keepdims=True)
    upd = jnp.concatenate([s0, s1, jnp.zeros((6, 128), F32)], axis=0)

    @pl.when(i == 0)
    def _():
        st_ref[...] = jnp.zeros_like(st_ref)

    st_ref[...] += upd


def _bn_affine(st_ref, g_ref, b_ref):
    mu = st_ref[0:1, :] / N
    var = st_ref[1:2, :] / N - mu * mu
    a = g_ref[...] / jnp.sqrt(var + EPS)
    return a, b_ref[...] - mu * a


def _tcn_body(y_ref, st_ref, g_ref, b_ref, h_ref):
    a, cb = _bn_affine(st_ref, g_ref, b_ref)
    h_ref[...] = a * y_ref[...] + cb


def _tcb_body(h1_ref, agg_ref, w1_ref, b1_ref, w2_ref, b2_ref, ib_ref,
              y2_ref, st2_ref, bnd_ref):
    i = pl.program_id(0)
    t2 = h1_ref[...] + agg_ref[0] + agg_ref[1]
    h = jnp.maximum(_dot(t2, w1_ref[...]) + b1_ref[...], 0.0)
    y = jnp.maximum(_dot(h, w2_ref[...]) + b2_ref[...], 0.0)
    y2_ref[...] = y
    rowid = i * R + lax.broadcasted_iota(I32, (R, 1), 0)
    valid = rowid < N
    ym = jnp.where(valid, y, 0.0)
    s0 = jnp.sum(ym, axis=0, keepdims=True)
    s1 = jnp.sum(ym * ym, axis=0, keepdims=True)
    upd = jnp.concatenate([s0, s1, jnp.zeros((6, 128), F32)], axis=0)

    ib = ib_ref[0, 0, :][:, None]
    gi = lax.broadcasted_iota(I32, (R, G), 1)
    one = jnp.ones((R, G), I32)
    zero = jnp.zeros((R, G), I32)
    lt = jnp.sum(jnp.where((ib < gi) & valid, one, zero), axis=0,
                 keepdims=True)
    le = jnp.sum(jnp.where((ib <= gi) & valid, one, zero), axis=0,
                 keepdims=True)
    bupd = jnp.concatenate([lt, le, jnp.zeros((6, 128), I32)], axis=0)

    @pl.when(i == 0)
    def _():
        st2_ref[...] = jnp.zeros_like(st2_ref)
        bnd_ref[...] = jnp.zeros_like(bnd_ref)

    st2_ref[...] += upd
    bnd_ref[...] += bupd


def _tcc_body(st2_ref, g2g_ref, g2b_ref, mm_ref,
              ge_ref, c1w_ref, c1b_ref, cbg_ref, cbb_ref, c2w_ref, c2b_ref,
              xd_ref, xc_ref):
    a2, c2 = _bn_affine(st2_ref, g2g_ref, g2b_ref)
    xd1 = mm_ref[0]
    xd2 = jnp.where(a2 > 0, a2 * mm_ref[1] + c2, a2 * mm_ref[2] + c2)
    xd_ref[...] = jnp.concatenate([xd1, xd2], axis=1)

    xc = jnp.tanh(_dot(ge_ref[...], c1w_ref[...]) + c1b_ref[...])
    mu = jnp.mean(xc, axis=0, keepdims=True)
    var = jnp.mean(xc * xc, axis=0, keepdims=True) - mu * mu
    xb = (xc - mu) / jnp.sqrt(var + EPS) * cbg_ref[...] + cbb_ref[...]
    xc_ref[...] = jnp.maximum(_dot(xb, c2w_ref[...]) + c2b_ref[...], 0.0)


def _full(shape):
    return pl.BlockSpec(shape, lambda *_: tuple(0 for _ in shape))


_tca = pl.pallas_call(
    _tca_body,
    grid=(NBLK,),
    in_specs=[
        pl.BlockSpec((R, 128), lambda i: (i, 0)),
        pl.BlockSpec((2, R, 128), lambda i: (0, i, 0)),
        _full((128, 128)), _full((1, 128)), _full((128, 128)), _full((1, 128)),
    ],
    out_specs=[
        pl.BlockSpec((R, 128), lambda i: (i, 0)),
        _full((8, 128)),
    ],
    out_shape=[
        jax.ShapeDtypeStruct((NPAD, 128), F32),
        jax.ShapeDtypeStruct((8, 128), F32),
    ],
)

_tcn = pl.pallas_call(
    _tcn_body,
    grid=(NBLK,),
    in_specs=[
        pl.BlockSpec((R, 128), lambda i: (i, 0)),
        _full((8, 128)), _full((1, 128)), _full((1, 128)),
    ],
    out_specs=pl.BlockSpec((R, 128), lambda i: (i, 0)),
    out_shape=jax.ShapeDtypeStruct((NPAD, 128), F32),
)

_tcb = pl.pallas_call(
    _tcb_body,
    grid=(NBLK,),
    in_specs=[
        pl.BlockSpec((R, 128), lambda i: (i, 0)),
        pl.BlockSpec((2, R, 128), lambda i: (0, i, 0)),
        _full((128, 128)), _full((1, 128)), _full((128, 128)), _full((1, 128)),
        pl.BlockSpec((1, 1, R), lambda i: (i, 0, 0)),
    ],
    out_specs=[
        pl.BlockSpec((R, 128), lambda i: (i, 0)),
        _full((8, 128)),
        _full((8, 128)),
    ],
    out_shape=[
        jax.ShapeDtypeStruct((NPAD, 128), F32),
        jax.ShapeDtypeStruct((8, 128), F32),
        jax.ShapeDtypeStruct((8, 128), I32),
    ],
)

_tcc = pl.pallas_call(
    _tcc_body,
    in_specs=[
        _full((8, 128)), _full((1, 128)), _full((1, 128)),
        _full((3, G, 128)),
        _full((G, 1024)), _full((1024, 128)), _full((1, 128)),
        _full((1, 128)), _full((1, 128)), _full((128, 128)), _full((1, 128)),
    ],
    out_specs=[_full((G, 256)), _full((G, 128))],
    out_shape=[
        jax.ShapeDtypeStruct((G, 256), F32),
        jax.ShapeDtypeStruct((G, 128), F32),
    ],
)


def kernel(drug_feature, drug_adj, ibatch, gexpr_data,
           g1_W1, g1_b1, g1_W2, g1_b2, g1_gamma, g1_beta,
           g2_W1, g2_b1, g2_W2, g2_b2, g2_gamma, g2_beta,
           c1_W, c1_b, cbn_gamma, cbn_beta, c2_W, c2_b):
    src = drug_adj[0].astype(I32)
    dst = drug_adj[1].astype(I32)
    src_p = jnp.concatenate([src, jnp.zeros((EPAD - E,), I32)])
    dst_p = jnp.concatenate(
        [dst, N + (jnp.arange(EPAD - E, dtype=I32) % (NACC - N))])

    x_pad = jnp.zeros((NPAD, 128), F32).at[:N].set(drug_feature)

    agg1 = _agg(drug_feature, src_p, dst_p)

    r2 = lambda v: v.reshape(1, 128)
    y1, st1 = _tca(x_pad, agg1, g1_W1, r2(g1_b1), g1_W2, r2(g1_b2))

    h1 = _tcn(y1, st1, r2(g1_gamma), r2(g1_beta))

    agg2 = _agg(h1, src_p, dst_p)

    ib_pad = jnp.concatenate(
        [ibatch.astype(I32), jnp.full((NPAD - N,), G, I32)]).reshape(NBLK, 1, R)
    y2, st2, bnd = _tcb(h1, agg2, g2_W1, r2(g2_b1), g2_W2, r2(g2_b2), ib_pad)

    bw = (jnp.zeros((NW, 16), I32)
          .at[:, 0:4].set(bnd[0].reshape(NW, 4))
          .at[:, 8:12].set(bnd[1].reshape(NW, 4)))
    mm = _segmax(h1, y2, bw.reshape(-1))

    ge_pad = jnp.zeros((G, 1024), F32).at[:, :DIM_CELL].set(gexpr_data)
    c1w_pad = jnp.zeros((1024, 128), F32).at[:DIM_CELL].set(c1_W)

    x_drug, x_cell = _tcc(st2, r2(g2_gamma), r2(g2_beta), mm,
                          ge_pad, c1w_pad, r2(c1_b),
                          r2(cbn_gamma), r2(cbn_beta), c2_W, r2(c2_b))
    return (x_drug, x_cell)


# speedup vs baseline: 1.3801x; 1.3782x over previous
import jax
import jax.numpy as jnp
from jax import lax
from jax.experimental import pallas as pl
from jax.experimental.pallas import tpu as pltpu
from jax.experimental.pallas import tpu_sc as plsc

N = 10000
E = 320000
D = 128
G = 128
DIM_CELL = 954

NW = 32
BIGCH = 128
NBCH = 79
EPW = NBCH * BIGCH
EPAD = NW * EPW

R = 1024
NBLK = 10
NPAD = NBLK * R
RPT = 632
NACC = 16 * RPT

F32 = jnp.float32
I32 = jnp.int32
EPS = 1e-5


def _mesh():
    return plsc.VectorSubcoreMesh(core_axis_name="c", subcore_axis_name="s",
                                  num_cores=2, num_subcores=16)


def _make_agg():
    out_type = jax.ShapeDtypeStruct((2, NPAD, 128), F32)
    scratch = [
        pltpu.VMEM((BIGCH,), I32),
        pltpu.VMEM((BIGCH,), I32),
        pltpu.VMEM((BIGCH, 128), F32),
        pltpu.VMEM_SHARED((NACC, 128), F32),
        pltpu.SemaphoreType.DMA,
    ]

    def body(x_hbm, src_hbm, dst_hbm, agg_out, sidx, didx, rows, acc, sem):
        c = lax.axis_index("c")
        s = lax.axis_index("s")
        w = c * 16 + s

        zv = jnp.zeros((16,), F32)

        def zrow(i, _):
            for cc in range(8):
                rows[i, pl.ds(cc * 16, 16)] = zv
            return 0

        lax.fori_loop(0, BIGCH, zrow, 0)

        base = s * RPT
        nfull = RPT // BIGCH
        remr = RPT - nfull * BIGCH
        for j in range(nfull):
            pltpu.sync_copy(rows, acc.at[pl.ds(base + j * BIGCH, BIGCH)])
        pltpu.sync_copy(rows.at[pl.ds(0, remr)],
                        acc.at[pl.ds(base + nfull * BIGCH, remr)])

        plsc.subcore_barrier()

        def edge_chunk(t, _):
            eb = (w * NBCH + t) * BIGCH
            pltpu.sync_copy(src_hbm.at[pl.ds(eb, BIGCH)], sidx)
            pltpu.sync_copy(dst_hbm.at[pl.ds(eb, BIGCH)], didx)
            pltpu.async_copy(x_hbm.at[sidx], rows, sem).wait()
            pltpu.sync_copy(rows, acc.at[didx], add=True)
            return 0

        lax.fori_loop(0, NBCH, edge_chunk, 0)

        plsc.subcore_barrier()

        ob = s * RPT
        pltpu.sync_copy(acc.at[pl.ds(ob, RPT)], agg_out.at[c, pl.ds(ob, RPT)])

    return pl.kernel(body, out_type=out_type, mesh=_mesh(),
                     scratch_types=scratch)


_agg = _make_agg()


def _segmax(h1, y2, bounds):
    out_type = jax.ShapeDtypeStruct((3 * G * 128,), F32)
    scratch = [
        pltpu.VMEM((16 * NW,), I32),
        pltpu.VMEM((16,), I32),
        pltpu.VMEM((16, 128), F32),
        pltpu.VMEM((16, 128), F32),
        pltpu.VMEM((4, 128), F32),
        pltpu.SemaphoreType.DMA,
    ]

    def body(h1_hbm, y2_hbm, bnd_hbm, out_hbm, bnd_v, idx_v, rb1, rb2, accb,
             sem):
        c = lax.axis_index("c")
        s = lax.axis_index("s")
        w = c * 16 + s
        pltpu.sync_copy(bnd_hbm, bnd_v)
        bvec = bnd_v[pl.ds(w * 16, 16)]
        lanes = lax.broadcasted_iota(I32, (16,), 0)

        for j in range(4):
            g = w * 4 + j
            st = bvec[j]
            en = bvec[8 + j]
            nch = (en - st + 15) // 16

            carry0 = tuple([jnp.full((16,), -jnp.inf, F32)] * 16
                           + [jnp.full((16,), jnp.inf, F32)] * 8)

            def chunk(jj, carry, st=st, en=en):
                iv = jnp.minimum(jnp.full((16,), st, I32) + jj * 16 + lanes,
                                 jnp.full((16,), en - 1, I32))
                idx_v[...] = iv
                pltpu.async_copy(h1_hbm.at[idx_v], rb1, sem).wait()
                pltpu.async_copy(y2_hbm.at[idx_v], rb2, sem).wait()
                out = list(carry)
                for k in range(16):
                    for cc in range(8):
                        v1 = rb1[k, pl.ds(cc * 16, 16)]
                        v2 = rb2[k, pl.ds(cc * 16, 16)]
                        out[cc] = jnp.maximum(out[cc], v1)
                        out[8 + cc] = jnp.maximum(out[8 + cc], v2)
                        out[16 + cc] = jnp.minimum(out[16 + cc], v2)
                return tuple(out)

            res = lax.fori_loop(0, nch, chunk, carry0)
            for cc in range(8):
                accb[0, pl.ds(cc * 16, 16)] = res[cc]
                accb[1, pl.ds(cc * 16, 16)] = res[8 + cc]
                accb[2, pl.ds(cc * 16, 16)] = res[16 + cc]
            for a in range(3):
                pltpu.sync_copy(accb.at[a],
                                out_hbm.at[pl.ds((a * G + g) * 128, 128)])

    fn = pl.kernel(body, out_type=out_type, mesh=_mesh(),
                   scratch_types=scratch)
    return fn(h1, y2, bounds).reshape(3, G, 128)


def _dot(a, b):
    return jnp.dot(a, b, preferred_element_type=F32,
                   precision=lax.Precision.HIGHEST)


def _tca_body(x_ref, agg_ref, w1_ref, b1_ref, w2_ref, b2_ref, y_ref, st_ref):
    i = pl.program_id(0)
    t = x_ref[...] + agg_ref[0] + agg_ref[1]
    h = jnp.maximum(_dot(t, w1_ref[...]) + b1_ref[...], 0.0)
    y = jnp.maximum(_dot(h, w2_ref[...]) + b2_ref[...], 0.0)
    y_ref[...] = y
    rowid = i * R + lax.broadcasted_iota(I32, (R, 1), 0)
    ym = jnp.where(rowid < N, y, 0.0)
    s0 = jnp.sum(ym, axis=0, keepdims=True)
    s1 = jnp.sum(ym * ym, axis=0, keepdims=True)
    upd = jnp.concatenate([s0, s1, jnp.zeros((6, 128), F32)], axis=0)

    @pl.when(i == 0)
    def _():
        st_ref[...] = jnp.zeros_like(st_ref)

    st_ref[...] += upd


def _bn_affine(st_ref, g_ref, b_ref):
    mu = st_ref[0:1, :] / N
    var = st_ref[1:2, :] / N - mu * mu
    a = g_ref[...] / jnp.sqrt(var + EPS)
    return a, b_ref[...] - mu * a


def _tcn_body(y_ref, st_ref, g_ref, b_ref, h_ref):
    a, cb = _bn_affine(st_ref, g_ref, b_ref)
    h_ref[...] = a * y_ref[...] + cb


def _tcb_body(h1_ref, agg_ref, w1_ref, b1_ref, w2_ref, b2_ref, ib_ref,
              y2_ref, st2_ref, bnd_ref):
    i = pl.program_id(0)
    t2 = h1_ref[...] + agg_ref[0] + agg_ref[1]
    h = jnp.maximum(_dot(t2, w1_ref[...]) + b1_ref[...], 0.0)
    y = jnp.maximum(_dot(h, w2_ref[...]) + b2_ref[...], 0.0)
    y2_ref[...] = y
    rowid = i * R + lax.broadcasted_iota(I32, (R, 1), 0)
    valid = rowid < N
    ym = jnp.where(valid, y, 0.0)
    s0 = jnp.sum(ym, axis=0, keepdims=True)
    s1 = jnp.sum(ym * ym, axis=0, keepdims=True)
    upd = jnp.concatenate([s0, s1, jnp.zeros((6, 128), F32)], axis=0)

    ib = ib_ref[0, 0, :][:, None]
    gi = lax.broadcasted_iota(I32, (R, G), 1)
    one = jnp.ones((R, G), I32)
    zero = jnp.zeros((R, G), I32)
    lt = jnp.sum(jnp.where((ib < gi) & valid, one, zero), axis=0,
                 keepdims=True)
    le = jnp.sum(jnp.where((ib <= gi) & valid, one, zero), axis=0,
                 keepdims=True)
    bupd = jnp.concatenate([lt, le, jnp.zeros((6, 128), I32)], axis=0)

    @pl.when(i == 0)
    def _():
        st2_ref[...] = jnp.zeros_like(st2_ref)
        bnd_ref[...] = jnp.zeros_like(bnd_ref)

    st2_ref[...] += upd
    bnd_ref[...] += bupd


def _tcc_body(st2_ref, g2g_ref, g2b_ref, mm_ref,
              ge_ref, c1w_ref, c1b_ref, cbg_ref, cbb_ref, c2w_ref, c2b_ref,
              xd_ref, xc_ref):
    a2, c2 = _bn_affine(st2_ref, g2g_ref, g2b_ref)
    xd1 = mm_ref[0]
    xd2 = jnp.where(a2 > 0, a2 * mm_ref[1] + c2, a2 * mm_ref[2] + c2)
    xd_ref[...] = jnp.concatenate([xd1, xd2], axis=1)

    xc = jnp.tanh(_dot(ge_ref[...], c1w_ref[...]) + c1b_ref[...])
    mu = jnp.mean(xc, axis=0, keepdims=True)
    var = jnp.mean(xc * xc, axis=0, keepdims=True) - mu * mu
    xb = (xc - mu) / jnp.sqrt(var + EPS) * cbg_ref[...] + cbb_ref[...]
    xc_ref[...] = jnp.maximum(_dot(xb, c2w_ref[...]) + c2b_ref[...], 0.0)


def _full(shape):
    return pl.BlockSpec(shape, lambda *_: tuple(0 for _ in shape))


_tca = pl.pallas_call(
    _tca_body,
    grid=(NBLK,),
    in_specs=[
        pl.BlockSpec((R, 128), lambda i: (i, 0)),
        pl.BlockSpec((2, R, 128), lambda i: (0, i, 0)),
        _full((128, 128)), _full((1, 128)), _full((128, 128)), _full((1, 128)),
    ],
    out_specs=[
        pl.BlockSpec((R, 128), lambda i: (i, 0)),
        _full((8, 128)),
    ],
    out_shape=[
        jax.ShapeDtypeStruct((NPAD, 128), F32),
        jax.ShapeDtypeStruct((8, 128), F32),
    ],
)

_tcn = pl.pallas_call(
    _tcn_body,
    grid=(NBLK,),
    in_specs=[
        pl.BlockSpec((R, 128), lambda i: (i, 0)),
        _full((8, 128)), _full((1, 128)), _full((1, 128)),
    ],
    out_specs=pl.BlockSpec((R, 128), lambda i: (i, 0)),
    out_shape=jax.ShapeDtypeStruct((NPAD, 128), F32),
)

_tcb = pl.pallas_call(
    _tcb_body,
    grid=(NBLK,),
    in_specs=[
        pl.BlockSpec((R, 128), lambda i: (i, 0)),
        pl.BlockSpec((2, R, 128), lambda i: (0, i, 0)),
        _full((128, 128)), _full((1, 128)), _full((128, 128)), _full((1, 128)),
        pl.BlockSpec((1, 1, R), lambda i: (i, 0, 0)),
    ],
    out_specs=[
        pl.BlockSpec((R, 128), lambda i: (i, 0)),
        _full((8, 128)),
        _full((8, 128)),
    ],
    out_shape=[
        jax.ShapeDtypeStruct((NPAD, 128), F32),
        jax.ShapeDtypeStruct((8, 128), F32),
        jax.ShapeDtypeStruct((8, 128), I32),
    ],
)

_tcc = pl.pallas_call(
    _tcc_body,
    in_specs=[
        _full((8, 128)), _full((1, 128)), _full((1, 128)),
        _full((3, G, 128)),
        _full((G, 1024)), _full((1024, 128)), _full((1, 128)),
        _full((1, 128)), _full((1, 128)), _full((128, 128)), _full((1, 128)),
    ],
    out_specs=[_full((G, 256)), _full((G, 128))],
    out_shape=[
        jax.ShapeDtypeStruct((G, 256), F32),
        jax.ShapeDtypeStruct((G, 128), F32),
    ],
)


def kernel(drug_feature, drug_adj, ibatch, gexpr_data,
           g1_W1, g1_b1, g1_W2, g1_b2, g1_gamma, g1_beta,
           g2_W1, g2_b1, g2_W2, g2_b2, g2_gamma, g2_beta,
           c1_W, c1_b, cbn_gamma, cbn_beta, c2_W, c2_b):
    src = drug_adj[0].astype(I32)
    dst = drug_adj[1].astype(I32)
    src_p = jnp.concatenate([src, jnp.zeros((EPAD - E,), I32)])
    dst_p = jnp.concatenate(
        [dst, N + (jnp.arange(EPAD - E, dtype=I32) % (NACC - N))])

    x_pad = jnp.zeros((NPAD, 128), F32).at[:N].set(drug_feature)

    agg1 = _agg(drug_feature, src_p, dst_p)

    r2 = lambda v: v.reshape(1, 128)
    y1, st1 = _tca(x_pad, agg1, g1_W1, r2(g1_b1), g1_W2, r2(g1_b2))

    h1 = _tcn(y1, st1, r2(g1_gamma), r2(g1_beta))

    agg2 = _agg(h1, src_p, dst_p)

    ib_pad = jnp.concatenate(
        [ibatch.astype(I32), jnp.full((NPAD - N,), G, I32)]).reshape(NBLK, 1, R)
    y2, st2, bnd = _tcb(h1, agg2, g2_W1, r2(g2_b1), g2_W2, r2(g2_b2), ib_pad)

    bw = (jnp.zeros((NW, 16), I32)
          .at[:, 0:4].set(bnd[0].reshape(NW, 4))
          .at[:, 8:12].set(bnd[1].reshape(NW, 4)))
    mm = _segmax(h1, y2, bw.reshape(-1))

    ge_pad = jnp.zeros((G, 1024), F32).at[:, :DIM_CELL].set(gexpr_data)
    c1w_pad = jnp.zeros((1024, 128), F32).at[:DIM_CELL].set(c1_W)

    x_drug, x_cell = _tcc(st2, r2(g2_gamma), r2(g2_beta), mm,
                          ge_pad, c1w_pad, r2(c1_b),
                          r2(cbn_gamma), r2(cbn_beta), c2_W, r2(c2_b))
    return (x_drug, x_cell)


# speedup vs baseline: 2.0300x; 1.4709x over previous
import jax
import jax.numpy as jnp
from jax import lax
from jax.experimental import pallas as pl
from jax.experimental.pallas import tpu as pltpu
from jax.experimental.pallas import tpu_sc as plsc

N = 10000
E = 320000
D = 128
G = 128
DIM_CELL = 954

NW = 32
BIGCH = 128
NBCH = 79
EPW = NBCH * BIGCH
EPAD = NW * EPW

R = 1024
NBLK = 10
NPAD = NBLK * R
RPT = 632
NACC = 16 * RPT

F32 = jnp.float32
I32 = jnp.int32
EPS = 1e-5


def _mesh():
    return plsc.VectorSubcoreMesh(core_axis_name="c", subcore_axis_name="s",
                                  num_cores=2, num_subcores=16)


def _make_agg():
    out_type = jax.ShapeDtypeStruct((2, NPAD, 128), F32)
    scratch = [
        pltpu.VMEM((BIGCH,), I32),
        pltpu.VMEM((BIGCH,), I32),
        pltpu.VMEM((BIGCH, 128), F32),
        pltpu.VMEM_SHARED((NACC, 128), F32),
        pltpu.SemaphoreType.DMA,
    ]

    def body(x_hbm, src_hbm, dst_hbm, agg_out, sidx, didx, rows, acc, sem):
        c = lax.axis_index("c")
        s = lax.axis_index("s")
        w = c * 16 + s

        zv = jnp.zeros((16,), F32)

        def zrow(i, _):
            for cc in range(8):
                rows[i, pl.ds(cc * 16, 16)] = zv
            return 0

        lax.fori_loop(0, BIGCH, zrow, 0)

        base = s * RPT
        nfull = RPT // BIGCH
        remr = RPT - nfull * BIGCH
        for j in range(nfull):
            pltpu.sync_copy(rows, acc.at[pl.ds(base + j * BIGCH, BIGCH)])
        pltpu.sync_copy(rows.at[pl.ds(0, remr)],
                        acc.at[pl.ds(base + nfull * BIGCH, remr)])

        plsc.subcore_barrier()

        def edge_chunk(t, _):
            eb = (w * NBCH + t) * BIGCH
            pltpu.sync_copy(src_hbm.at[pl.ds(eb, BIGCH)], sidx)
            pltpu.sync_copy(dst_hbm.at[pl.ds(eb, BIGCH)], didx)
            pltpu.async_copy(x_hbm.at[sidx], rows, sem).wait()
            pltpu.sync_copy(rows, acc.at[didx], add=True)
            return 0

        lax.fori_loop(0, NBCH, edge_chunk, 0)

        plsc.subcore_barrier()

        ob = s * RPT
        pltpu.sync_copy(acc.at[pl.ds(ob, RPT)], agg_out.at[c, pl.ds(ob, RPT)])

    return pl.kernel(body, out_type=out_type, mesh=_mesh(),
                     scratch_types=scratch)


_agg = _make_agg()


def _segmax(h1, y2, bounds):
    out_type = jax.ShapeDtypeStruct((3 * G * 128,), F32)
    scratch = [
        pltpu.VMEM((16 * NW,), I32),
        pltpu.VMEM((16,), I32),
        pltpu.VMEM((16, 128), F32),
        pltpu.VMEM((16, 128), F32),
        pltpu.VMEM((4, 128), F32),
        pltpu.SemaphoreType.DMA,
    ]

    def body(h1_hbm, y2_hbm, bnd_hbm, out_hbm, bnd_v, idx_v, rb1, rb2, accb,
             sem):
        c = lax.axis_index("c")
        s = lax.axis_index("s")
        w = c * 16 + s
        pltpu.sync_copy(bnd_hbm, bnd_v)
        bvec = bnd_v[pl.ds(w * 16, 16)]
        lanes = lax.broadcasted_iota(I32, (16,), 0)

        for j in range(4):
            g = w * 4 + j
            st = bvec[j]
            en = bvec[8 + j]
            nch = (en - st + 15) // 16

            carry0 = tuple([jnp.full((16,), -jnp.inf, F32)] * 16
                           + [jnp.full((16,), jnp.inf, F32)] * 8)

            def chunk(jj, carry, st=st, en=en):
                iv = jnp.minimum(jnp.full((16,), st, I32) + jj * 16 + lanes,
                                 jnp.full((16,), en - 1, I32))
                idx_v[...] = iv
                pltpu.async_copy(h1_hbm.at[idx_v], rb1, sem).wait()
                pltpu.async_copy(y2_hbm.at[idx_v], rb2, sem).wait()
                out = list(carry)
                for k in range(16):
                    for cc in range(8):
                        v1 = rb1[k, pl.ds(cc * 16, 16)]
                        v2 = rb2[k, pl.ds(cc * 16, 16)]
                        out[cc] = jnp.maximum(out[cc], v1)
                        out[8 + cc] = jnp.maximum(out[8 + cc], v2)
                        out[16 + cc] = jnp.minimum(out[16 + cc], v2)
                return tuple(out)

            res = lax.fori_loop(0, nch, chunk, carry0)
            for cc in range(8):
                accb[0, pl.ds(cc * 16, 16)] = res[cc]
                accb[1, pl.ds(cc * 16, 16)] = res[8 + cc]
                accb[2, pl.ds(cc * 16, 16)] = res[16 + cc]
            for a in range(3):
                pltpu.sync_copy(accb.at[a],
                                out_hbm.at[pl.ds((a * G + g) * 128, 128)])

    fn = pl.kernel(body, out_type=out_type, mesh=_mesh(),
                   scratch_types=scratch)
    return fn(h1, y2, bounds).reshape(3, G, 128)


def _dot(a, b):
    return jnp.dot(a, b, preferred_element_type=F32,
                   precision=lax.Precision.HIGHEST)


def _tca_body(x_ref, agg_ref, w1_ref, b1_ref, w2_ref, b2_ref, y_ref, st_ref):
    i = pl.program_id(0)
    t = x_ref[...] + agg_ref[0] + agg_ref[1]
    h = jnp.maximum(_dot(t, w1_ref[...]) + b1_ref[...], 0.0)
    y = jnp.maximum(_dot(h, w2_ref[...]) + b2_ref[...], 0.0)
    y_ref[...] = y
    rowid = i * R + lax.broadcasted_iota(I32, (R, 1), 0)
    ym = jnp.where(rowid < N, y, 0.0)
    s0 = jnp.sum(ym, axis=0, keepdims=True)
    s1 = jnp.sum(ym * ym, axis=0, keepdims=True)
    upd = jnp.concatenate([s0, s1, jnp.zeros((6, 128), F32)], axis=0)

    @pl.when(i == 0)
    def _():
        st_ref[...] = jnp.zeros_like(st_ref)

    st_ref[...] += upd


def _bn_affine(st_ref, g_ref, b_ref):
    mu = st_ref[0:1, :] / N
    var = st_ref[1:2, :] / N - mu * mu
    a = g_ref[...] / jnp.sqrt(var + EPS)
    return a, b_ref[...] - mu * a


def _tcn_body(y_ref, st_ref, g_ref, b_ref, h_ref):
    a, cb = _bn_affine(st_ref, g_ref, b_ref)
    h_ref[...] = a * y_ref[...] + cb


def _tcb_body(h1_ref, agg_ref, w1_ref, b1_ref, w2_ref, b2_ref, ib_ref,
              y2_ref, st2_ref, bnd_ref):
    i = pl.program_id(0)
    t2 = h1_ref[...] + agg_ref[0] + agg_ref[1]
    h = jnp.maximum(_dot(t2, w1_ref[...]) + b1_ref[...], 0.0)
    y = jnp.maximum(_dot(h, w2_ref[...]) + b2_ref[...], 0.0)
    y2_ref[...] = y
    rowid = i * R + lax.broadcasted_iota(I32, (R, 1), 0)
    valid = rowid < N
    ym = jnp.where(valid, y, 0.0)
    s0 = jnp.sum(ym, axis=0, keepdims=True)
    s1 = jnp.sum(ym * ym, axis=0, keepdims=True)
    upd = jnp.concatenate([s0, s1, jnp.zeros((6, 128), F32)], axis=0)

    ib = ib_ref[0, 0, :][:, None]
    gi = lax.broadcasted_iota(I32, (R, G), 1)
    one = jnp.ones((R, G), I32)
    zero = jnp.zeros((R, G), I32)
    lt = jnp.sum(jnp.where((ib < gi) & valid, one, zero), axis=0,
                 keepdims=True)
    le = jnp.sum(jnp.where((ib <= gi) & valid, one, zero), axis=0,
                 keepdims=True)
    bupd = jnp.concatenate([lt, le, jnp.zeros((6, 128), I32)], axis=0)

    @pl.when(i == 0)
    def _():
        st2_ref[...] = jnp.zeros_like(st2_ref)
        bnd_ref[...] = jnp.zeros_like(bnd_ref)

    st2_ref[...] += upd
    bnd_ref[...] += bupd


def _tcc_body(st2_ref, g2g_ref, g2b_ref, mm_ref,
              ge_ref, c1w_ref, c1b_ref, cbg_ref, cbb_ref, c2w_ref, c2b_ref,
              xd_ref, xc_ref):
    a2, c2 = _bn_affine(st2_ref, g2g_ref, g2b_ref)
    xd1 = mm_ref[0]
    xd2 = jnp.where(a2 > 0, a2 * mm_ref[1] + c2, a2 * mm_ref[2] + c2)
    xd_ref[...] = jnp.concatenate([xd1, xd2], axis=1)

    xc = jnp.tanh(_dot(ge_ref[...], c1w_ref[...]) + c1b_ref[...])
    mu = jnp.mean(xc, axis=0, keepdims=True)
    var = jnp.mean(xc * xc, axis=0, keepdims=True) - mu * mu
    xb = (xc - mu) / jnp.sqrt(var + EPS) * cbg_ref[...] + cbb_ref[...]
    xc_ref[...] = jnp.maximum(_dot(xb, c2w_ref[...]) + c2b_ref[...], 0.0)


def _full(shape):
    return pl.BlockSpec(shape, lambda *_: tuple(0 for _ in shape))


_tca = pl.pallas_call(
    _tca_body,
    grid=(NBLK,),
    in_specs=[
        pl.BlockSpec((R, 128), lambda i: (i, 0)),
        pl.BlockSpec((2, R, 128), lambda i: (0, i, 0)),
        _full((128, 128)), _full((1, 128)), _full((128, 128)), _full((1, 128)),
    ],
    out_specs=[
        pl.BlockSpec((R, 128), lambda i: (i, 0)),
        _full((8, 128)),
    ],
    out_shape=[
        jax.ShapeDtypeStruct((NPAD, 128), F32),
        jax.ShapeDtypeStruct((8, 128), F32),
    ],
)

_tcn = pl.pallas_call(
    _tcn_body,
    grid=(NBLK,),
    in_specs=[
        pl.BlockSpec((R, 128), lambda i: (i, 0)),
        _full((8, 128)), _full((1, 128)), _full((1, 128)),
    ],
    out_specs=pl.BlockSpec((R, 128), lambda i: (i, 0)),
    out_shape=jax.ShapeDtypeStruct((NPAD, 128), F32),
)

_tcb = pl.pallas_call(
    _tcb_body,
    grid=(NBLK,),
    in_specs=[
        pl.BlockSpec((R, 128), lambda i: (i, 0)),
        pl.BlockSpec((2, R, 128), lambda i: (0, i, 0)),
        _full((128, 128)), _full((1, 128)), _full((128, 128)), _full((1, 128)),
        pl.BlockSpec((1, 1, R), lambda i: (i, 0, 0)),
    ],
    out_specs=[
        pl.BlockSpec((R, 128), lambda i: (i, 0)),
        _full((8, 128)),
        _full((8, 128)),
    ],
    out_shape=[
        jax.ShapeDtypeStruct((NPAD, 128), F32),
        jax.ShapeDtypeStruct((8, 128), F32),
        jax.ShapeDtypeStruct((8, 128), I32),
    ],
)

_tcc = pl.pallas_call(
    _tcc_body,
    in_specs=[
        _full((8, 128)), _full((1, 128)), _full((1, 128)),
        _full((3, G, 128)),
        _full((G, 1024)), _full((1024, 128)), _full((1, 128)),
        _full((1, 128)), _full((1, 128)), _full((128, 128)), _full((1, 128)),
    ],
    out_specs=[_full((G, 256)), _full((G, 128))],
    out_shape=[
        jax.ShapeDtypeStruct((G, 256), F32),
        jax.ShapeDtypeStruct((G, 128), F32),
    ],
)


def kernel(drug_feature, drug_adj, ibatch, gexpr_data,
           g1_W1, g1_b1, g1_W2, g1_b2, g1_gamma, g1_beta,
           g2_W1, g2_b1, g2_W2, g2_b2, g2_gamma, g2_beta,
           c1_W, c1_b, cbn_gamma, cbn_beta, c2_W, c2_b):
    src = drug_adj[0].astype(I32)
    dst = drug_adj[1].astype(I32)
    pad_iota = jnp.arange(EPAD - E, dtype=I32)
    src_p = jnp.concatenate([src, pad_iota % 128])
    dst_p = jnp.concatenate([dst, N + (pad_iota % (NACC - N))])

    x_pad = jnp.zeros((NPAD, 128), F32).at[:N].set(drug_feature)

    agg1 = _agg(drug_feature, src_p, dst_p)

    r2 = lambda v: v.reshape(1, 128)
    y1, st1 = _tca(x_pad, agg1, g1_W1, r2(g1_b1), g1_W2, r2(g1_b2))

    h1 = _tcn(y1, st1, r2(g1_gamma), r2(g1_beta))

    agg2 = _agg(h1, src_p, dst_p)

    ib_pad = jnp.concatenate(
        [ibatch.astype(I32), jnp.full((NPAD - N,), G, I32)]).reshape(NBLK, 1, R)
    y2, st2, bnd = _tcb(h1, agg2, g2_W1, r2(g2_b1), g2_W2, r2(g2_b2), ib_pad)

    bw = (jnp.zeros((NW, 16), I32)
          .at[:, 0:4].set(bnd[0].reshape(NW, 4))
          .at[:, 8:12].set(bnd[1].reshape(NW, 4)))
    mm = _segmax(h1, y2, bw.reshape(-1))

    ge_pad = jnp.zeros((G, 1024), F32).at[:, :DIM_CELL].set(gexpr_data)
    c1w_pad = jnp.zeros((1024, 128), F32).at[:DIM_CELL].set(c1_W)

    x_drug, x_cell = _tcc(st2, r2(g2_gamma), r2(g2_beta), mm,
                          ge_pad, c1w_pad, r2(c1_b),
                          r2(cbn_gamma), r2(cbn_beta), c2_W, r2(c2_b))
    return (x_drug, x_cell)


# speedup vs baseline: 3.3434x; 1.6470x over previous
import jax
import jax.numpy as jnp
from jax import lax
from jax.experimental import pallas as pl
from jax.experimental.pallas import tpu as pltpu
from jax.experimental.pallas import tpu_sc as plsc

N = 10000
E = 320000
D = 128
G = 128
DIM_CELL = 954

NW = 32
BIGCH = 128
NBCH = 80
EPW = NBCH * BIGCH
EPAD = NW * EPW

R = 1024
NBLK = 10
NPAD = NBLK * R
RPT = 632
NACC = 16 * RPT

F32 = jnp.float32
I32 = jnp.int32
EPS = 1e-5


def _mesh():
    return plsc.VectorSubcoreMesh(core_axis_name="c", subcore_axis_name="s",
                                  num_cores=2, num_subcores=16)


def _make_agg():
    out_type = jax.ShapeDtypeStruct((2, NPAD, 128), F32)
    scratch = [
        pltpu.VMEM((BIGCH,), I32), pltpu.VMEM((BIGCH,), I32),
        pltpu.VMEM((BIGCH,), I32), pltpu.VMEM((BIGCH,), I32),
        pltpu.VMEM((BIGCH, 128), F32),
        pltpu.VMEM((BIGCH, 128), F32),
        pltpu.VMEM_SHARED((NACC, 128), F32),
        pltpu.SemaphoreType.DMA, pltpu.SemaphoreType.DMA,
        pltpu.SemaphoreType.DMA, pltpu.SemaphoreType.DMA,
    ]

    def body(x_hbm, src_hbm, dst_hbm, agg_out, sa, sb, da, db, ra, rb, acc,
             gsa, gsb, isa, isb):
        c = lax.axis_index("c")
        s = lax.axis_index("s")
        w = c * 16 + s
        sx = (sa, sb)
        dx = (da, db)
        rx = (ra, rb)
        gs = (gsa, gsb)
        ism = (isa, isb)

        zv = jnp.zeros((16,), F32)

        def zrow(i, _):
            for cc in range(8):
                ra[i, pl.ds(cc * 16, 16)] = zv
            return 0

        lax.fori_loop(0, BIGCH, zrow, 0)

        base = s * RPT
        nfull = RPT // BIGCH
        remr = RPT - nfull * BIGCH
        for j in range(nfull):
            pltpu.sync_copy(ra, acc.at[pl.ds(base + j * BIGCH, BIGCH)])
        pltpu.sync_copy(ra.at[pl.ds(0, remr)],
                        acc.at[pl.ds(base + nfull * BIGCH, remr)])

        plsc.subcore_barrier()

        ebase = w * NBCH * BIGCH
        pltpu.sync_copy(src_hbm.at[pl.ds(ebase, BIGCH)], sa)
        pltpu.sync_copy(dst_hbm.at[pl.ds(ebase, BIGCH)], da)
        pltpu.async_copy(x_hbm.at[sa], ra, gsa)
        pltpu.async_copy(src_hbm.at[pl.ds(ebase + BIGCH, BIGCH)], sb, isb)
        pltpu.async_copy(dst_hbm.at[pl.ds(ebase + BIGCH, BIGCH)], db, isb)

        def outer(i, _):
            for u in range(2):
                t = i * 2 + u
                b, o = u, 1 - u
                nonlast = i < NBCH // 2 - 1

                def fire_gather(b=b, o=o):
                    pltpu.make_async_copy(src_hbm.at[pl.ds(0, BIGCH)],
                                          sx[o], ism[o]).wait()
                    pltpu.make_async_copy(dst_hbm.at[pl.ds(0, BIGCH)],
                                          dx[o], ism[o]).wait()
                    pltpu.async_copy(x_hbm.at[sx[o]], rx[o], gs[o])

                if u == 0:
                    fire_gather()
                else:
                    pl.when(nonlast)(fire_gather)

                pltpu.make_async_copy(x_hbm.at[sx[b]], rx[b], gs[b]).wait()
                pltpu.sync_copy(rx[b], acc.at[dx[b]], add=True)

                def fire_idx(t=t, b=b):
                    nb = ebase + (t + 2) * BIGCH
                    pltpu.async_copy(src_hbm.at[pl.ds(nb, BIGCH)],
                                     sx[b], ism[b])
                    pltpu.async_copy(dst_hbm.at[pl.ds(nb, BIGCH)],
                                     dx[b], ism[b])

                pl.when(nonlast)(fire_idx)
            return 0

        lax.fori_loop(0, NBCH // 2, outer, 0)

        plsc.subcore_barrier()

        ob = s * RPT
        pltpu.sync_copy(acc.at[pl.ds(ob, RPT)], agg_out.at[c, pl.ds(ob, RPT)])

    return pl.kernel(body, out_type=out_type, mesh=_mesh(),
                     scratch_types=scratch)


_agg = _make_agg()


def _segmax(h1, y2, bounds):
    out_type = jax.ShapeDtypeStruct((3 * G * 128,), F32)
    scratch = [
        pltpu.VMEM((16 * NW,), I32),
        pltpu.VMEM((16,), I32),
        pltpu.VMEM((16, 128), F32),
        pltpu.VMEM((16, 128), F32),
        pltpu.VMEM((4, 128), F32),
        pltpu.SemaphoreType.DMA,
    ]

    def body(h1_hbm, y2_hbm, bnd_hbm, out_hbm, bnd_v, idx_v, rb1, rb2, accb,
             sem):
        c = lax.axis_index("c")
        s = lax.axis_index("s")
        w = c * 16 + s
        pltpu.sync_copy(bnd_hbm, bnd_v)
        bvec = bnd_v[pl.ds(w * 16, 16)]
        lanes = lax.broadcasted_iota(I32, (16,), 0)

        for j in range(4):
            g = w * 4 + j
            st = bvec[j]
            en = bvec[8 + j]
            nch = (en - st + 15) // 16

            carry0 = tuple([jnp.full((16,), -jnp.inf, F32)] * 16
                           + [jnp.full((16,), jnp.inf, F32)] * 8)

            def chunk(jj, carry, st=st, en=en):
                iv = jnp.minimum(jnp.full((16,), st, I32) + jj * 16 + lanes,
                                 jnp.full((16,), en - 1, I32))
                idx_v[...] = iv
                pltpu.async_copy(h1_hbm.at[idx_v], rb1, sem).wait()
                pltpu.async_copy(y2_hbm.at[idx_v], rb2, sem).wait()
                out = list(carry)
                for k in range(16):
                    for cc in range(8):
                        v1 = rb1[k, pl.ds(cc * 16, 16)]
                        v2 = rb2[k, pl.ds(cc * 16, 16)]
                        out[cc] = jnp.maximum(out[cc], v1)
                        out[8 + cc] = jnp.maximum(out[8 + cc], v2)
                        out[16 + cc] = jnp.minimum(out[16 + cc], v2)
                return tuple(out)

            res = lax.fori_loop(0, nch, chunk, carry0)
            for cc in range(8):
                accb[0, pl.ds(cc * 16, 16)] = res[cc]
                accb[1, pl.ds(cc * 16, 16)] = res[8 + cc]
                accb[2, pl.ds(cc * 16, 16)] = res[16 + cc]
            for a in range(3):
                pltpu.sync_copy(accb.at[a],
                                out_hbm.at[pl.ds((a * G + g) * 128, 128)])

    fn = pl.kernel(body, out_type=out_type, mesh=_mesh(),
                   scratch_types=scratch)
    return fn(h1, y2, bounds).reshape(3, G, 128)


def _dot(a, b):
    return jnp.dot(a, b, preferred_element_type=F32,
                   precision=lax.Precision.HIGHEST)


def _tca_body(x_ref, agg_ref, w1_ref, b1_ref, w2_ref, b2_ref, y_ref, st_ref):
    i = pl.program_id(0)
    t = x_ref[...] + agg_ref[0] + agg_ref[1]
    h = jnp.maximum(_dot(t, w1_ref[...]) + b1_ref[...], 0.0)
    y = jnp.maximum(_dot(h, w2_ref[...]) + b2_ref[...], 0.0)
    y_ref[...] = y
    rowid = i * R + lax.broadcasted_iota(I32, (R, 1), 0)
    ym = jnp.where(rowid < N, y, 0.0)
    s0 = jnp.sum(ym, axis=0, keepdims=True)
    s1 = jnp.sum(ym * ym, axis=0, keepdims=True)
    upd = jnp.concatenate([s0, s1, jnp.zeros((6, 128), F32)], axis=0)

    @pl.when(i == 0)
    def _():
        st_ref[...] = jnp.zeros_like(st_ref)

    st_ref[...] += upd


def _bn_affine(st_ref, g_ref, b_ref):
    mu = st_ref[0:1, :] / N
    var = st_ref[1:2, :] / N - mu * mu
    a = g_ref[...] / jnp.sqrt(var + EPS)
    return a, b_ref[...] - mu * a


def _tcn_body(y_ref, st_ref, g_ref, b_ref, h_ref):
    a, cb = _bn_affine(st_ref, g_ref, b_ref)
    h_ref[...] = a * y_ref[...] + cb


def _tcb_body(h1_ref, agg_ref, w1_ref, b1_ref, w2_ref, b2_ref, ib_ref,
              y2_ref, st2_ref, bnd_ref):
    i = pl.program_id(0)
    t2 = h1_ref[...] + agg_ref[0] + agg_ref[1]
    h = jnp.maximum(_dot(t2, w1_ref[...]) + b1_ref[...], 0.0)
    y = jnp.maximum(_dot(h, w2_ref[...]) + b2_ref[...], 0.0)
    y2_ref[...] = y
    rowid = i * R + lax.broadcasted_iota(I32, (R, 1), 0)
    valid = rowid < N
    ym = jnp.where(valid, y, 0.0)
    s0 = jnp.sum(ym, axis=0, keepdims=True)
    s1 = jnp.sum(ym * ym, axis=0, keepdims=True)
    upd = jnp.concatenate([s0, s1, jnp.zeros((6, 128), F32)], axis=0)

    ib = ib_ref[0, 0, :][:, None]
    gi = lax.broadcasted_iota(I32, (R, G), 1)
    one = jnp.ones((R, G), I32)
    zero = jnp.zeros((R, G), I32)
    lt = jnp.sum(jnp.where((ib < gi) & valid, one, zero), axis=0,
                 keepdims=True)
    le = jnp.sum(jnp.where((ib <= gi) & valid, one, zero), axis=0,
                 keepdims=True)
    bupd = jnp.concatenate([lt, le, jnp.zeros((6, 128), I32)], axis=0)

    @pl.when(i == 0)
    def _():
        st2_ref[...] = jnp.zeros_like(st2_ref)
        bnd_ref[...] = jnp.zeros_like(bnd_ref)

    st2_ref[...] += upd
    bnd_ref[...] += bupd


def _tcc_body(st2_ref, g2g_ref, g2b_ref, mm_ref,
              ge_ref, c1w_ref, c1b_ref, cbg_ref, cbb_ref, c2w_ref, c2b_ref,
              xd_ref, xc_ref):
    a2, c2 = _bn_affine(st2_ref, g2g_ref, g2b_ref)
    xd1 = mm_ref[0]
    xd2 = jnp.where(a2 > 0, a2 * mm_ref[1] + c2, a2 * mm_ref[2] + c2)
    xd_ref[...] = jnp.concatenate([xd1, xd2], axis=1)

    xc = jnp.tanh(_dot(ge_ref[...], c1w_ref[...]) + c1b_ref[...])
    mu = jnp.mean(xc, axis=0, keepdims=True)
    var = jnp.mean(xc * xc, axis=0, keepdims=True) - mu * mu
    xb = (xc - mu) / jnp.sqrt(var + EPS) * cbg_ref[...] + cbb_ref[...]
    xc_ref[...] = jnp.maximum(_dot(xb, c2w_ref[...]) + c2b_ref[...], 0.0)


def _full(shape):
    return pl.BlockSpec(shape, lambda *_: tuple(0 for _ in shape))


_tca = pl.pallas_call(
    _tca_body,
    grid=(NBLK,),
    in_specs=[
        pl.BlockSpec((R, 128), lambda i: (i, 0)),
        pl.BlockSpec((2, R, 128), lambda i: (0, i, 0)),
        _full((128, 128)), _full((1, 128)), _full((128, 128)), _full((1, 128)),
    ],
    out_specs=[
        pl.BlockSpec((R, 128), lambda i: (i, 0)),
        _full((8, 128)),
    ],
    out_shape=[
        jax.ShapeDtypeStruct((NPAD, 128), F32),
        jax.ShapeDtypeStruct((8, 128), F32),
    ],
)

_tcn = pl.pallas_call(
    _tcn_body,
    grid=(NBLK,),
    in_specs=[
        pl.BlockSpec((R, 128), lambda i: (i, 0)),
        _full((8, 128)), _full((1, 128)), _full((1, 128)),
    ],
    out_specs=pl.BlockSpec((R, 128), lambda i: (i, 0)),
    out_shape=jax.ShapeDtypeStruct((NPAD, 128), F32),
)

_tcb = pl.pallas_call(
    _tcb_body,
    grid=(NBLK,),
    in_specs=[
        pl.BlockSpec((R, 128), lambda i: (i, 0)),
        pl.BlockSpec((2, R, 128), lambda i: (0, i, 0)),
        _full((128, 128)), _full((1, 128)), _full((128, 128)), _full((1, 128)),
        pl.BlockSpec((1, 1, R), lambda i: (i, 0, 0)),
    ],
    out_specs=[
        pl.BlockSpec((R, 128), lambda i: (i, 0)),
        _full((8, 128)),
        _full((8, 128)),
    ],
    out_shape=[
        jax.ShapeDtypeStruct((NPAD, 128), F32),
        jax.ShapeDtypeStruct((8, 128), F32),
        jax.ShapeDtypeStruct((8, 128), I32),
    ],
)

_tcc = pl.pallas_call(
    _tcc_body,
    in_specs=[
        _full((8, 128)), _full((1, 128)), _full((1, 128)),
        _full((3, G, 128)),
        _full((G, 1024)), _full((1024, 128)), _full((1, 128)),
        _full((1, 128)), _full((1, 128)), _full((128, 128)), _full((1, 128)),
    ],
    out_specs=[_full((G, 256)), _full((G, 128))],
    out_shape=[
        jax.ShapeDtypeStruct((G, 256), F32),
        jax.ShapeDtypeStruct((G, 128), F32),
    ],
)


def kernel(drug_feature, drug_adj, ibatch, gexpr_data,
           g1_W1, g1_b1, g1_W2, g1_b2, g1_gamma, g1_beta,
           g2_W1, g2_b1, g2_W2, g2_b2, g2_gamma, g2_beta,
           c1_W, c1_b, cbn_gamma, cbn_beta, c2_W, c2_b):
    src = drug_adj[0].astype(I32)
    dst = drug_adj[1].astype(I32)
    pad_iota = jnp.arange(EPAD - E, dtype=I32)
    src_p = jnp.concatenate([src, pad_iota % 128])
    dst_p = jnp.concatenate([dst, N + (pad_iota % (NACC - N))])

    x_pad = jnp.zeros((NPAD, 128), F32).at[:N].set(drug_feature)

    agg1 = _agg(drug_feature, src_p, dst_p)

    r2 = lambda v: v.reshape(1, 128)
    y1, st1 = _tca(x_pad, agg1, g1_W1, r2(g1_b1), g1_W2, r2(g1_b2))

    h1 = _tcn(y1, st1, r2(g1_gamma), r2(g1_beta))

    agg2 = _agg(h1, src_p, dst_p)

    ib_pad = jnp.concatenate(
        [ibatch.astype(I32), jnp.full((NPAD - N,), G, I32)]).reshape(NBLK, 1, R)
    y2, st2, bnd = _tcb(h1, agg2, g2_W1, r2(g2_b1), g2_W2, r2(g2_b2), ib_pad)

    bw = (jnp.zeros((NW, 16), I32)
          .at[:, 0:4].set(bnd[0].reshape(NW, 4))
          .at[:, 8:12].set(bnd[1].reshape(NW, 4)))
    mm = _segmax(h1, y2, bw.reshape(-1))

    ge_pad = jnp.zeros((G, 1024), F32).at[:, :DIM_CELL].set(gexpr_data)
    c1w_pad = jnp.zeros((1024, 128), F32).at[:DIM_CELL].set(c1_W)

    x_drug, x_cell = _tcc(st2, r2(g2_gamma), r2(g2_beta), mm,
                          ge_pad, c1w_pad, r2(c1_b),
                          r2(cbn_gamma), r2(cbn_beta), c2_W, r2(c2_b))
    return (x_drug, x_cell)
